# baseline (speedup 1.0000x reference)
import functools

import jax
import jax.numpy as jnp
from jax import lax
from jax.experimental import pallas as pl
from jax.experimental.pallas import tpu as pltpu
from jax.experimental.pallas import tpu_sc as plsc

GRID = 64
TS = GRID * GRID * GRID
TS_PAD = TS + 128
C = 128
NK = 27

NC = 2
NS = 16
NW = NC * NS

CHUNK = 128
W_CHUNKS = 13
CH = W_CHUNKS * CHUNK
NP = NW * CH
S_CHUNKS = NP // (NS * CHUNK)
SCH = S_CHUNKS * CHUNK

BM = 512
NB = NP // BM


def _sc_gather_kernel(cx_hbm, cy_hbm, cz_hbm, fe_hbm, g_hbm, table_hbm,
                      cx1, cy1, cz1, lin1, vals1,
                      cx2, cy2, cz2, lin2, nkv, valv, srcv, idxv,
                      rows, fill, sem_t, sem_r):
    c = lax.axis_index("c")
    s = lax.axis_index("s")
    wid = c * NS + s
    c_off = c * TS_PAD

    def fill_body(t, _):
        fill[pl.ds(t * 16, 16)] = jnp.full((16,), -1, jnp.int32)
        return 0
    lax.fori_loop(0, 128, fill_body, 0)
    seg = TS_PAD // NS
    start = c * TS_PAD + s * seg
    for m in range(seg // 2048):
        pltpu.sync_copy(fill, table_hbm.at[pl.ds(start + m * 2048, 2048)])
    rem = seg % 2048
    if rem:
        pltpu.sync_copy(fill.at[pl.ds(0, rem)],
                        table_hbm.at[pl.ds(start + (seg // 2048) * 2048, rem)])
    plsc.subcore_barrier()

    base1 = s * SCH
    pltpu.sync_copy(cx_hbm.at[pl.ds(base1, SCH)], cx1)
    pltpu.sync_copy(cy_hbm.at[pl.ds(base1, SCH)], cy1)
    pltpu.sync_copy(cz_hbm.at[pl.ds(base1, SCH)], cz1)

    def lin_body(j, _):
        def t_body(t, _):
            sl = pl.ds(j * CHUNK + t * 16, 16)
            lin = (cx1[sl] * GRID + cy1[sl]) * GRID + cz1[sl] + c_off
            lin1[j, pl.ds(t * 16, 16)] = lin
            vals1[sl] = base1 + j * CHUNK + t * 16 + lax.iota(jnp.int32, 16)
            return 0
        lax.fori_loop(0, CHUNK // 16, t_body, 0)
        return 0
    lax.fori_loop(0, S_CHUNKS, lin_body, 0)

    def scat_body(j, _):
        pltpu.async_copy(vals1.at[pl.ds(j * CHUNK, CHUNK)],
                         table_hbm.at[lin1.at[j]], sem_t).wait()
        return 0
    lax.fori_loop(0, S_CHUNKS, scat_body, 0)
    plsc.subcore_barrier()

    base2 = wid * CH
    pltpu.sync_copy(cx_hbm.at[pl.ds(base2, CH)], cx2)
    pltpu.sync_copy(cy_hbm.at[pl.ds(base2, CH)], cy2)
    pltpu.sync_copy(cz_hbm.at[pl.ds(base2, CH)], cz2)

    def lin2_body(t, _):
        sl = pl.ds(t * 16, 16)
        lin2[sl] = (cx2[sl] * GRID + cy2[sl]) * GRID + cz2[sl] + c_off
        return 0
    lax.fori_loop(0, CH // 16, lin2_body, 0)

    n_rows = fe_hbm.shape[0] - 8

    def k_body(k, _):
        dx = k // 9 - 1
        dy = (k // 3) % 3 - 1
        dz = k % 3 - 1
        d = dx * (GRID * GRID) + dy * GRID + dz

        def nk_body(t, _):
            sl = pl.ds(t * 16, 16)
            ncx = cx2[sl] + dx
            ncy = cy2[sl] + dy
            ncz = cz2[sl] + dz
            ok = ((ncx >= 0) & (ncx < GRID) & (ncy >= 0) & (ncy < GRID)
                  & (ncz >= 0) & (ncz < GRID))
            nk = lin2[sl] + d
            nk = jnp.minimum(jnp.maximum(nk, c_off), c_off + TS - 1)
            nkv[sl] = nk
            valv[sl] = jnp.where(ok, 0, -1)
            return 0
        lax.fori_loop(0, CH // 16, nk_body, 0)

        def look_body(j, _):
            pltpu.async_copy(table_hbm.at[nkv.at[pl.ds(j * CHUNK, CHUNK)]],
                             srcv.at[pl.ds(j * CHUNK, CHUNK)], sem_t).wait()
            return 0
        lax.fori_loop(0, W_CHUNKS, look_body, 0)

        def fin_body(t, _):
            sl = pl.ds(t * 16, 16)
            src = srcv[sl]
            hit = (valv[sl] == 0) & (src >= 0)
            idxv[sl] = jnp.where(hit, src, n_rows)
            return 0
        lax.fori_loop(0, CH // 16, fin_body, 0)

        def row_body(j, _):
            pltpu.async_copy(fe_hbm.at[idxv.at[pl.ds(j * CHUNK, CHUNK)]],
                             rows, sem_r).wait()
            pltpu.sync_copy(rows, g_hbm.at[k, pl.ds(base2 + j * CHUNK, CHUNK)])
            return 0
        lax.fori_loop(0, W_CHUNKS, row_body, 0)
        return 0
    lax.fori_loop(0, NK, k_body, 0)


def _sc_gather(cxp, cyp, czp, feats_ext):
    kfn = functools.partial(
        pl.kernel,
        out_type=(
            jax.ShapeDtypeStruct((NK, NP, C), jnp.float32),
            jax.ShapeDtypeStruct((NC * TS_PAD,), jnp.int32),
        ),
        mesh=plsc.VectorSubcoreMesh(core_axis_name="c", subcore_axis_name="s"),
        scratch_types=[
            pltpu.VMEM((SCH,), jnp.int32),
            pltpu.VMEM((SCH,), jnp.int32),
            pltpu.VMEM((SCH,), jnp.int32),
            pltpu.VMEM((S_CHUNKS, CHUNK), jnp.int32),
            pltpu.VMEM((SCH,), jnp.int32),
            pltpu.VMEM((CH,), jnp.int32),
            pltpu.VMEM((CH,), jnp.int32),
            pltpu.VMEM((CH,), jnp.int32),
            pltpu.VMEM((CH,), jnp.int32),
            pltpu.VMEM((CH,), jnp.int32),
            pltpu.VMEM((CH,), jnp.int32),
            pltpu.VMEM((CH,), jnp.int32),
            pltpu.VMEM((CH,), jnp.int32),
            pltpu.VMEM((CHUNK, C), jnp.float32),
            pltpu.VMEM((2048,), jnp.int32),
            pltpu.SemaphoreType.DMA,
            pltpu.SemaphoreType.DMA,
        ],
        compiler_params=pltpu.CompilerParams(use_tc_tiling_on_sc=False),
    )(_sc_gather_kernel)
    g, _ = kfn(cxp, cyp, czp, feats_ext)
    return g


def _tc_matmul_body(n_valid, g_ref, w_ref, out_ref, p_ref):
    bi = pl.program_id(0)
    k = pl.program_id(1)
    contrib = jnp.dot(g_ref[0], w_ref[0], preferred_element_type=jnp.float32)

    @pl.when(k == 0)
    def _():
        out_ref[...] = contrib

    @pl.when(k > 0)
    def _():
        out_ref[...] = out_ref[...] + contrib

    @pl.when(k == NK - 1)
    def _():
        acc = out_ref[...]
        gidx = bi * BM + lax.broadcasted_iota(jnp.int32, (BM, C), 0)
        masked = jnp.where(gidx < n_valid, acc, 0.0)
        ssum = jnp.sum(masked, axis=0, keepdims=True)
        ssq = jnp.sum(masked * masked, axis=0, keepdims=True)
        p_ref[...] = jnp.concatenate(
            [ssum, ssq, jnp.zeros((6, C), jnp.float32)], axis=0)[None]


def _tc_matmul(g, w, n_valid):
    return pl.pallas_call(
        functools.partial(_tc_matmul_body, n_valid),
        grid=(NB, NK),
        in_specs=[
            pl.BlockSpec((1, BM, C), lambda bi, k: (k, bi, 0)),
            pl.BlockSpec((1, C, C), lambda bi, k: (k, 0, 0)),
        ],
        out_specs=[
            pl.BlockSpec((BM, C), lambda bi, k: (bi, 0)),
            pl.BlockSpec((1, 8, C), lambda bi, k: (bi, 0, 0)),
        ],
        out_shape=[
            jax.ShapeDtypeStruct((NP, C), jnp.float32),
            jax.ShapeDtypeStruct((NB, 8, C), jnp.float32),
        ],
        compiler_params=pltpu.CompilerParams(
            dimension_semantics=("arbitrary", "arbitrary")),
    )(g, w)


def _tc_stats_body(n_valid, p_ref, ga_ref, be_ref, out_ref):
    ps = p_ref[...]
    ssum = jnp.sum(ps[:, 0, :], axis=0, keepdims=True)
    ssq = jnp.sum(ps[:, 1, :], axis=0, keepdims=True)
    inv_n = 1.0 / n_valid
    mean = ssum * inv_n
    var = ssq * inv_n - mean * mean
    scale = ga_ref[...] * lax.rsqrt(var + 1e-6)
    shift = be_ref[...] - mean * scale
    out_ref[...] = jnp.concatenate(
        [scale, shift, jnp.zeros((6, C), jnp.float32)], axis=0)


def _tc_stats(partials, gamma2, beta2, n_valid):
    return pl.pallas_call(
        functools.partial(_tc_stats_body, float(n_valid)),
        out_shape=jax.ShapeDtypeStruct((8, C), jnp.float32),
    )(partials, gamma2, beta2)


def _tc_apply_body(o_ref, sc_ref, y_ref):
    x = o_ref[...]
    y = x * sc_ref[0:1, :] + sc_ref[1:2, :]
    y_ref[...] = jnp.maximum(y, 0.0)


def _tc_apply(out_full, sc):
    return pl.pallas_call(
        _tc_apply_body,
        grid=(NB,),
        in_specs=[
            pl.BlockSpec((BM, C), lambda bi: (bi, 0)),
            pl.BlockSpec((8, C), lambda bi: (0, 0)),
        ],
        out_specs=pl.BlockSpec((BM, C), lambda bi: (bi, 0)),
        out_shape=jax.ShapeDtypeStruct((NP, C), jnp.float32),
    )(out_full, sc)


def kernel(feats, coords, W, bn_gamma, bn_beta):
    n = feats.shape[0]
    pad = NP - n
    cxp = jnp.concatenate([coords[:, 0], jnp.full((pad,), GRID, jnp.int32)])
    cyp = jnp.concatenate([coords[:, 1], jnp.zeros((pad,), jnp.int32)])
    czp = jnp.concatenate([coords[:, 2], jnp.zeros((pad,), jnp.int32)])
    feats_ext = jnp.concatenate([feats, jnp.zeros((8, C), jnp.float32)], axis=0)

    g = _sc_gather(cxp, cyp, czp, feats_ext)
    out_full, partials = _tc_matmul(g, W, n)
    sc = _tc_stats(partials, bn_gamma.reshape(1, C), bn_beta.reshape(1, C), n)
    y = _tc_apply(out_full, sc)
    return y[:n]


# speedup vs baseline: 1.0010x; 1.0010x over previous
import functools

import jax
import jax.numpy as jnp
from jax import lax
from jax.experimental import pallas as pl
from jax.experimental.pallas import tpu as pltpu
from jax.experimental.pallas import tpu_sc as plsc

GRID = 64
TS = GRID * GRID * GRID
TS_PAD = TS + 128
C = 128
NK = 27

NC = 2
NS = 16
NW = NC * NS

CHUNK = 128
W_CHUNKS = 13
CH = W_CHUNKS * CHUNK
NP = NW * CH
S_CHUNKS = NP // (NS * CHUNK)
SCH = S_CHUNKS * CHUNK

BM = 512
NB = NP // BM

RB = 32
NSTR = 16
_g = 0
ROW_GROUPS = []
while _g < CH:
    _n = min(NSTR, (CH - _g) // RB)
    ROW_GROUPS.append((_g, _n))
    _g += _n * RB
ROW_GROUPS = tuple(ROW_GROUPS)


def _sc_gather_kernel(cx_hbm, cy_hbm, cz_hbm, fe_hbm, g_hbm, table_hbm,
                      cx1, cy1, cz1, lin1, vals1,
                      cx2, cy2, cz2, lin2, nkv, valv, srcv, idxv,
                      rows, fill, sem_t, sem_r):
    c = lax.axis_index("c")
    s = lax.axis_index("s")
    wid = c * NS + s
    c_off = c * TS_PAD

    def fill_body(t, _):
        fill[pl.ds(t * 16, 16)] = jnp.full((16,), -1, jnp.int32)
        return 0
    lax.fori_loop(0, 128, fill_body, 0)
    seg = TS_PAD // NS
    start = c * TS_PAD + s * seg
    for m in range(seg // 2048):
        pltpu.sync_copy(fill, table_hbm.at[pl.ds(start + m * 2048, 2048)])
    rem = seg % 2048
    if rem:
        pltpu.sync_copy(fill.at[pl.ds(0, rem)],
                        table_hbm.at[pl.ds(start + (seg // 2048) * 2048, rem)])
    plsc.subcore_barrier()

    base1 = s * SCH
    pltpu.sync_copy(cx_hbm.at[pl.ds(base1, SCH)], cx1)
    pltpu.sync_copy(cy_hbm.at[pl.ds(base1, SCH)], cy1)
    pltpu.sync_copy(cz_hbm.at[pl.ds(base1, SCH)], cz1)

    def lin_body(j, _):
        def t_body(t, _):
            sl = pl.ds(j * CHUNK + t * 16, 16)
            lin = (cx1[sl] * GRID + cy1[sl]) * GRID + cz1[sl] + c_off
            lin1[j, pl.ds(t * 16, 16)] = lin
            vals1[sl] = base1 + j * CHUNK + t * 16 + lax.iota(jnp.int32, 16)
            return 0
        lax.fori_loop(0, CHUNK // 16, t_body, 0)
        return 0
    lax.fori_loop(0, S_CHUNKS, lin_body, 0)

    def scat_body(j, _):
        pltpu.async_copy(vals1.at[pl.ds(j * CHUNK, CHUNK)],
                         table_hbm.at[lin1.at[j]], sem_t).wait()
        return 0
    lax.fori_loop(0, S_CHUNKS, scat_body, 0)
    plsc.subcore_barrier()

    base2 = wid * CH
    pltpu.sync_copy(cx_hbm.at[pl.ds(base2, CH)], cx2)
    pltpu.sync_copy(cy_hbm.at[pl.ds(base2, CH)], cy2)
    pltpu.sync_copy(cz_hbm.at[pl.ds(base2, CH)], cz2)

    def lin2_body(t, _):
        sl = pl.ds(t * 16, 16)
        lin2[sl] = (cx2[sl] * GRID + cy2[sl]) * GRID + cz2[sl] + c_off
        return 0
    lax.fori_loop(0, CH // 16, lin2_body, 0)

    n_rows = fe_hbm.shape[0] - 8

    def k_body(k, _):
        dx = k // 9 - 1
        dy = (k // 3) % 3 - 1
        dz = k % 3 - 1
        d = dx * (GRID * GRID) + dy * GRID + dz

        def nk_body(t, _):
            sl = pl.ds(t * 16, 16)
            ncx = cx2[sl] + dx
            ncy = cy2[sl] + dy
            ncz = cz2[sl] + dz
            ok = ((ncx >= 0) & (ncx < GRID) & (ncy >= 0) & (ncy < GRID)
                  & (ncz >= 0) & (ncz < GRID))
            nk = lin2[sl] + d
            nk = jnp.minimum(jnp.maximum(nk, c_off), c_off + TS - 1)
            nkv[sl] = nk
            valv[sl] = jnp.where(ok, 0, -1)
            return 0
        lax.fori_loop(0, CH // 16, nk_body, 0)

        cps = [pltpu.async_copy(table_hbm.at[nkv.at[pl.ds(j * CHUNK, CHUNK)]],
                                srcv.at[pl.ds(j * CHUNK, CHUNK)], sem_t)
               for j in range(W_CHUNKS)]
        for cp in cps:
            cp.wait()

        def fin_body(t, _):
            sl = pl.ds(t * 16, 16)
            src = srcv[sl]
            hit = (valv[sl] == 0) & (src >= 0)
            idxv[sl] = jnp.where(hit, src, n_rows)
            return 0
        lax.fori_loop(0, CH // 16, fin_body, 0)

        for g0, nstr in ROW_GROUPS:
            cps = [pltpu.async_copy(
                       fe_hbm.at[idxv.at[pl.ds(g0 + m * RB, RB)]],
                       rows.at[m], sem_r)
                   for m in range(nstr)]
            for cp in cps:
                cp.wait()
            pltpu.sync_copy(
                rows.at[pl.ds(0, nstr)],
                g_hbm.at[k, pl.ds((base2 + g0) // RB, nstr)])
        return 0
    lax.fori_loop(0, NK, k_body, 0)


def _sc_gather(cxp, cyp, czp, feats_ext):
    kfn = functools.partial(
        pl.kernel,
        out_type=(
            jax.ShapeDtypeStruct((NK, NP // RB, RB, C), jnp.float32),
            jax.ShapeDtypeStruct((NC * TS_PAD,), jnp.int32),
        ),
        mesh=plsc.VectorSubcoreMesh(core_axis_name="c", subcore_axis_name="s"),
        scratch_types=[
            pltpu.VMEM((SCH,), jnp.int32),
            pltpu.VMEM((SCH,), jnp.int32),
            pltpu.VMEM((SCH,), jnp.int32),
            pltpu.VMEM((S_CHUNKS, CHUNK), jnp.int32),
            pltpu.VMEM((SCH,), jnp.int32),
            pltpu.VMEM((CH,), jnp.int32),
            pltpu.VMEM((CH,), jnp.int32),
            pltpu.VMEM((CH,), jnp.int32),
            pltpu.VMEM((CH,), jnp.int32),
            pltpu.VMEM((CH,), jnp.int32),
            pltpu.VMEM((CH,), jnp.int32),
            pltpu.VMEM((CH,), jnp.int32),
            pltpu.VMEM((CH,), jnp.int32),
            pltpu.VMEM((NSTR, RB, C), jnp.float32),
            pltpu.VMEM((2048,), jnp.int32),
            pltpu.SemaphoreType.DMA,
            pltpu.SemaphoreType.DMA,
        ],
        compiler_params=pltpu.CompilerParams(use_tc_tiling_on_sc=False),
    )(_sc_gather_kernel)
    g, _ = kfn(cxp, cyp, czp, feats_ext)
    return g


def _tc_matmul_body(n_valid, g_ref, w_ref, out_ref, p_ref):
    bi = pl.program_id(0)
    k = pl.program_id(1)
    contrib = jnp.dot(g_ref[...].reshape(BM, C), w_ref[0],
                      preferred_element_type=jnp.float32)

    @pl.when(k == 0)
    def _():
        out_ref[...] = contrib

    @pl.when(k > 0)
    def _():
        out_ref[...] = out_ref[...] + contrib

    @pl.when(k == NK - 1)
    def _():
        acc = out_ref[...]
        gidx = bi * BM + lax.broadcasted_iota(jnp.int32, (BM, C), 0)
        masked = jnp.where(gidx < n_valid, acc, 0.0)
        ssum = jnp.sum(masked, axis=0, keepdims=True)
        ssq = jnp.sum(masked * masked, axis=0, keepdims=True)
        p_ref[...] = jnp.concatenate(
            [ssum, ssq, jnp.zeros((6, C), jnp.float32)], axis=0)[None]


def _tc_matmul(g, w, n_valid):
    return pl.pallas_call(
        functools.partial(_tc_matmul_body, n_valid),
        grid=(NB, NK),
        in_specs=[
            pl.BlockSpec((1, BM // RB, RB, C), lambda bi, k: (k, bi, 0, 0)),
            pl.BlockSpec((1, C, C), lambda bi, k: (k, 0, 0)),
        ],
        out_specs=[
            pl.BlockSpec((BM, C), lambda bi, k: (bi, 0)),
            pl.BlockSpec((1, 8, C), lambda bi, k: (bi, 0, 0)),
        ],
        out_shape=[
            jax.ShapeDtypeStruct((NP, C), jnp.float32),
            jax.ShapeDtypeStruct((NB, 8, C), jnp.float32),
        ],
        compiler_params=pltpu.CompilerParams(
            dimension_semantics=("arbitrary", "arbitrary")),
    )(g, w)


def _tc_stats_body(n_valid, p_ref, ga_ref, be_ref, out_ref):
    ps = p_ref[...]
    ssum = jnp.sum(ps[:, 0, :], axis=0, keepdims=True)
    ssq = jnp.sum(ps[:, 1, :], axis=0, keepdims=True)
    inv_n = 1.0 / n_valid
    mean = ssum * inv_n
    var = ssq * inv_n - mean * mean
    scale = ga_ref[...] * lax.rsqrt(var + 1e-6)
    shift = be_ref[...] - mean * scale
    out_ref[...] = jnp.concatenate(
        [scale, shift, jnp.zeros((6, C), jnp.float32)], axis=0)


def _tc_stats(partials, gamma2, beta2, n_valid):
    return pl.pallas_call(
        functools.partial(_tc_stats_body, float(n_valid)),
        out_shape=jax.ShapeDtypeStruct((8, C), jnp.float32),
    )(partials, gamma2, beta2)


def _tc_apply_body(o_ref, sc_ref, y_ref):
    x = o_ref[...]
    y = x * sc_ref[0:1, :] + sc_ref[1:2, :]
    y_ref[...] = jnp.maximum(y, 0.0)


def _tc_apply(out_full, sc):
    return pl.pallas_call(
        _tc_apply_body,
        grid=(NB,),
        in_specs=[
            pl.BlockSpec((BM, C), lambda bi: (bi, 0)),
            pl.BlockSpec((8, C), lambda bi: (0, 0)),
        ],
        out_specs=pl.BlockSpec((BM, C), lambda bi: (bi, 0)),
        out_shape=jax.ShapeDtypeStruct((NP, C), jnp.float32),
    )(out_full, sc)


def kernel(feats, coords, W, bn_gamma, bn_beta):
    n = feats.shape[0]
    pad = NP - n
    cxp = jnp.concatenate([coords[:, 0], jnp.full((pad,), GRID, jnp.int32)])
    cyp = jnp.concatenate([coords[:, 1], jnp.zeros((pad,), jnp.int32)])
    czp = jnp.concatenate([coords[:, 2], jnp.zeros((pad,), jnp.int32)])
    feats_ext = jnp.concatenate([feats, jnp.zeros((8, C), jnp.float32)], axis=0)

    g = _sc_gather(cxp, cyp, czp, feats_ext)
    out_full, partials = _tc_matmul(g, W, n)
    sc = _tc_stats(partials, bn_gamma.reshape(1, C), bn_beta.reshape(1, C), n)
    y = _tc_apply(out_full, sc)
    return y[:n]


# speedup vs baseline: 32.7189x; 32.6871x over previous
import functools

import jax
import jax.numpy as jnp
from jax import lax
from jax.experimental import pallas as pl
from jax.experimental.pallas import tpu as pltpu
from jax.experimental.pallas import tpu_sc as plsc

GRID = 64
C = 128
NK = 27

NC = 2
NS = 16
NW = NC * NS

CHUNK = 128
W_CHUNKS = 13
CH = W_CHUNKS * CHUNK
NP = NW * CH

PY = 72
PX = 66 * PY
HALO = 4832

BM2 = 8192
NBLK = 39
NR_INT = NBLK * BM2
WIN = BM2 + 2 * HALO
ND = 331776

BM = 512
NB = NP // BM


MSET_ROWS = 256
MSET_N = ND // (NS * MSET_ROWS)


def _sc_scatter_kernel(cx_hbm, cy_hbm, cz_hbm, fp_hbm, d_hbm,
                       cxv, cyv, czv, linv, rowa, rowb, zbuf, sem, semz):
    c = lax.axis_index("c")
    s = lax.axis_index("s")
    base = (c * NS + s) * CH
    plane = c * ND

    def zb_body(r, _):
        def t_body(t, _):
            zbuf[r, pl.ds(t * 16, 16)] = jnp.zeros((16,), jnp.float32)
            return 0
        lax.fori_loop(0, C // 16, t_body, 0)
        return 0
    lax.fori_loop(0, MSET_ROWS, zb_body, 0)

    mstart = plane + s * (ND // NS)

    def ms_body(m, _):
        cps = [pltpu.async_copy(
                   zbuf, d_hbm.at[pl.ds(mstart + (m * 3 + i) * MSET_ROWS,
                                        MSET_ROWS)], semz)
               for i in range(3)]
        for cp in cps:
            cp.wait()
        return 0
    lax.fori_loop(0, MSET_N // 3, ms_body, 0)
    plsc.subcore_barrier()

    pltpu.sync_copy(cx_hbm.at[pl.ds(base, CH)], cxv)
    pltpu.sync_copy(cy_hbm.at[pl.ds(base, CH)], cyv)
    pltpu.sync_copy(cz_hbm.at[pl.ds(base, CH)], czv)

    def lin_body(j, _):
        def t_body(t, _):
            sl = pl.ds(j * CHUNK + t * 16, 16)
            lin = (cxv[sl] + 1) * PX + (cyv[sl] + 1) * PY + czv[sl] + 1
            linv[j, pl.ds(t * 16, 16)] = lin + (HALO + plane)
            return 0
        lax.fori_loop(0, CHUNK // 16, t_body, 0)
        return 0
    lax.fori_loop(0, W_CHUNKS, lin_body, 0)

    bufs = (rowa, rowb)
    cps = [None, None]
    for j in range(W_CHUNKS):
        b = j % 2
        if cps[b] is not None:
            cps[b].wait()
        pltpu.sync_copy(fp_hbm.at[pl.ds(base + j * CHUNK, CHUNK)], bufs[b])
        cps[b] = pltpu.async_copy(bufs[b], d_hbm.at[linv.at[j]], sem)
    cps[0].wait()
    cps[1].wait()


def _sc_scatter(cxp, cyp, czp, featsp):
    kfn = functools.partial(
        pl.kernel,
        out_type=jax.ShapeDtypeStruct((NC * ND, C), jnp.float32),
        mesh=plsc.VectorSubcoreMesh(core_axis_name="c", subcore_axis_name="s"),
        scratch_types=[
            pltpu.VMEM((CH,), jnp.int32),
            pltpu.VMEM((CH,), jnp.int32),
            pltpu.VMEM((CH,), jnp.int32),
            pltpu.VMEM((W_CHUNKS, CHUNK), jnp.int32),
            pltpu.VMEM((CHUNK, C), jnp.float32),
            pltpu.VMEM((CHUNK, C), jnp.float32),
            pltpu.VMEM((MSET_ROWS, C), jnp.float32),
            pltpu.SemaphoreType.DMA,
            pltpu.SemaphoreType.DMA,
        ],
    )(_sc_scatter_kernel)
    return kfn(cxp, cyp, czp, featsp)


RT = 248
RT2 = 512


def _tc_conv_body(d_ref, w_ref, o_ref, win, wsum, wm1, wp1, sem):
    bi = pl.program_id(0)
    cp0 = pltpu.make_async_copy(d_ref.at[pl.ds(bi * BM2, WIN + 8)],
                                wsum.at[pl.ds(8, WIN + 8)], sem)
    cp1 = pltpu.make_async_copy(d_ref.at[pl.ds(ND + bi * BM2, WIN + 8)],
                                win, sem)
    cp0.start()
    cp1.start()
    cp0.wait()
    cp1.wait()
    for q in range(WIN // RT):
        sl = pl.ds(8 + q * RT, RT)
        wsum[sl, :] = wsum[sl, :] + win[pl.ds(q * RT, RT), :]
    tl = pl.ds(8 + WIN, 8)
    wsum[tl, :] = wsum[tl, :] + win[pl.ds(WIN, 8), :]
    for q in range(WIN // RT):
        wm1[pl.ds(q * RT, RT), :] = wsum[pl.ds(q * RT + 9, RT), :]
        wp1[pl.ds(q * RT, RT), :] = wsum[pl.ds(q * RT + 7, RT), :]
    for p in range(BM2 // RT2):
        po = p * RT2
        osl = pl.ds(po, RT2)
        k = 0
        for dx in (-1, 0, 1):
            for dy in (-1, 0, 1):
                for dz in (-1, 0, 1):
                    off = HALO + dx * PX + dy * PY + po
                    if dz == 0:
                        lhs = wsum[pl.ds(off + 8, RT2), :]
                    elif dz == 1:
                        lhs = wm1[pl.ds(off, RT2), :]
                    else:
                        lhs = wp1[pl.ds(off, RT2), :]
                    contrib = jnp.dot(lhs, w_ref[k],
                                      preferred_element_type=jnp.float32)
                    if k == 0:
                        o_ref[osl, :] = contrib
                    else:
                        o_ref[osl, :] = o_ref[osl, :] + contrib
                    k += 1


def _tc_conv(d, w):
    return pl.pallas_call(
        _tc_conv_body,
        grid=(NBLK,),
        in_specs=[
            pl.BlockSpec(memory_space=pltpu.MemorySpace.HBM),
            pl.BlockSpec((NK, C, C), lambda bi: (0, 0, 0)),
        ],
        out_specs=pl.BlockSpec((BM2, C), lambda bi: (bi, 0)),
        out_shape=jax.ShapeDtypeStruct((NR_INT, C), jnp.float32),
        scratch_shapes=[
            pltpu.VMEM((WIN + 8, C), jnp.float32),
            pltpu.VMEM((WIN + 16, C), jnp.float32),
            pltpu.VMEM((WIN, C), jnp.float32),
            pltpu.VMEM((WIN, C), jnp.float32),
            pltpu.SemaphoreType.DMA,
        ],
        compiler_params=pltpu.CompilerParams(
            dimension_semantics=("arbitrary",)),
    )(d, w)


def _sc_gather_kernel(n_valid, cx_hbm, cy_hbm, cz_hbm, od_hbm,
                      out_hbm, part_hbm,
                      cxv, cyv, czv, linv, rowa, rowb, sumb, sem):
    c = lax.axis_index("c")
    s = lax.axis_index("s")
    wid = c * NS + s
    base = wid * CH

    pltpu.sync_copy(cx_hbm.at[pl.ds(base, CH)], cxv)
    pltpu.sync_copy(cy_hbm.at[pl.ds(base, CH)], cyv)
    pltpu.sync_copy(cz_hbm.at[pl.ds(base, CH)], czv)

    def lin_body(t, _):
        sl = pl.ds(t * 16, 16)
        linv[sl] = (cxv[sl] + 1) * PX + (cyv[sl] + 1) * PY + czv[sl] + 1
        return 0
    lax.fori_loop(0, CH // 16, lin_body, 0)

    def zero_body(r, _):
        def t_body(t, _):
            sumb[r, pl.ds(t * 16, 16)] = jnp.zeros((16,), jnp.float32)
            return 0
        lax.fori_loop(0, C // 16, t_body, 0)
        return 0
    lax.fori_loop(0, 8, zero_body, 0)

    bufs = (rowa, rowb)
    acc = tuple(jnp.zeros((16,), jnp.float32) for _ in range(16))
    cps = [None, None]
    cps[0] = pltpu.async_copy(od_hbm.at[linv.at[pl.ds(0, CHUNK)]], rowa, sem)
    for j in range(W_CHUNKS):
        buf = bufs[j % 2]
        cps[j % 2].wait()
        if j + 1 < W_CHUNKS:
            nbuf = bufs[(j + 1) % 2]
            cps[(j + 1) % 2] = pltpu.async_copy(
                od_hbm.at[linv.at[pl.ds((j + 1) * CHUNK, CHUNK)]], nbuf, sem)

        def row_body(r, a):
            gid = base + j * CHUNK + r
            m = jnp.where(gid < n_valid, 1.0, 0.0)
            new = []
            for t in range(8):
                x = buf[r, pl.ds(t * 16, 16)] * m
                new.append(a[t] + x)
                new.append(a[8 + t] + x * x)
            return tuple(new[::2]) + tuple(new[1::2])
        acc = lax.fori_loop(0, CHUNK, row_body, acc)
        pltpu.sync_copy(buf, out_hbm.at[pl.ds(base + j * CHUNK, CHUNK)])

    for t in range(8):
        sumb[0, pl.ds(t * 16, 16)] = acc[t]
        sumb[1, pl.ds(t * 16, 16)] = acc[8 + t]
    pltpu.sync_copy(sumb, part_hbm.at[wid])


def _sc_gather(cxp, cyp, czp, outdense, n_valid):
    kfn = functools.partial(
        pl.kernel,
        out_type=(
            jax.ShapeDtypeStruct((NP, C), jnp.float32),
            jax.ShapeDtypeStruct((NW, 8, C), jnp.float32),
        ),
        mesh=plsc.VectorSubcoreMesh(core_axis_name="c", subcore_axis_name="s"),
        scratch_types=[
            pltpu.VMEM((CH,), jnp.int32),
            pltpu.VMEM((CH,), jnp.int32),
            pltpu.VMEM((CH,), jnp.int32),
            pltpu.VMEM((CH,), jnp.int32),
            pltpu.VMEM((CHUNK, C), jnp.float32),
            pltpu.VMEM((CHUNK, C), jnp.float32),
            pltpu.VMEM((8, C), jnp.float32),
            pltpu.SemaphoreType.DMA,
        ],
    )(functools.partial(_sc_gather_kernel, n_valid))
    return kfn(cxp, cyp, czp, outdense)


def _tc_stats_body(n_valid, p_ref, ga_ref, be_ref, out_ref):
    ps = p_ref[...]
    ssum = jnp.sum(ps[:, 0, :], axis=0, keepdims=True)
    ssq = jnp.sum(ps[:, 1, :], axis=0, keepdims=True)
    inv_n = 1.0 / n_valid
    mean = ssum * inv_n
    var = ssq * inv_n - mean * mean
    scale = ga_ref[...] * lax.rsqrt(var + 1e-6)
    shift = be_ref[...] - mean * scale
    out_ref[...] = jnp.concatenate(
        [scale, shift, jnp.zeros((6, C), jnp.float32)], axis=0)


def _tc_stats(partials, gamma2, beta2, n_valid):
    return pl.pallas_call(
        functools.partial(_tc_stats_body, float(n_valid)),
        out_shape=jax.ShapeDtypeStruct((8, C), jnp.float32),
    )(partials, gamma2, beta2)


def _tc_apply_body(o_ref, sc_ref, y_ref):
    x = o_ref[...]
    y = x * sc_ref[0:1, :] + sc_ref[1:2, :]
    y_ref[...] = jnp.maximum(y, 0.0)


def _tc_apply(out_full, sc):
    return pl.pallas_call(
        _tc_apply_body,
        grid=(NB,),
        in_specs=[
            pl.BlockSpec((BM, C), lambda bi: (bi, 0)),
            pl.BlockSpec((8, C), lambda bi: (0, 0)),
        ],
        out_specs=pl.BlockSpec((BM, C), lambda bi: (bi, 0)),
        out_shape=jax.ShapeDtypeStruct((NP, C), jnp.float32),
    )(out_full, sc)


def kernel(feats, coords, W, bn_gamma, bn_beta):
    n = feats.shape[0]
    pad = NP - n
    cxp = jnp.concatenate([coords[:, 0], jnp.full((pad,), GRID, jnp.int32)])
    cyp = jnp.concatenate([coords[:, 1], jnp.zeros((pad,), jnp.int32)])
    czp = jnp.concatenate([coords[:, 2], jnp.zeros((pad,), jnp.int32)])
    featsp = jnp.concatenate([feats, jnp.zeros((pad, C), jnp.float32)], axis=0)

    d = _sc_scatter(cxp, cyp, czp, featsp)
    outdense = _tc_conv(d, W)
    out_full, partials = _sc_gather(cxp, cyp, czp, outdense, n)
    sc = _tc_stats(partials, bn_gamma.reshape(1, C), bn_beta.reshape(1, C), n)
    y = _tc_apply(out_full, sc)
    return y[:n]
